# baseline (speedup 1.0000x reference)
import functools

import jax
import jax.numpy as jnp
from jax import lax
from jax.experimental import pallas as pl
from jax.experimental.pallas import tpu as pltpu
from jax.experimental.pallas import tpu_sc as plsc

_LANES = 16
_KTOK = 128


_IDXW = _KTOK + _KTOK * _LANES


@functools.partial(jax.jit, static_argnums=(5, 6, 7, 8))
def _sc_embed(idx2, msp, token_table, pe0, dif, T, D, L, NW):
  G = (T // _KTOK) // NW
  mesh = plsc.VectorSubcoreMesh(core_axis_name="c", subcore_axis_name="s")

  @functools.partial(
      pl.kernel,
      mesh=mesh,
      out_type=jax.ShapeDtypeStruct((T, D), jnp.float32),
      scratch_types=[
          pltpu.VMEM((2, 1, _KTOK), jnp.int32),
          pltpu.VMEM((2, _KTOK, _LANES), jnp.float32),
          pltpu.VMEM((2, _KTOK, D), jnp.float32),
          pltpu.VMEM((L, D), jnp.float32),
          pltpu.VMEM((1, D), jnp.float32),
      ] + [pltpu.SemaphoreType.DMA] * 8,
  )
  def k(idx_hbm, msp_hbm, tab_hbm, pe0_hbm, dif_hbm, out_hbm, idx_v, msp_v,
        rows_v, pe_v, dif_v, s_i0, s_i1, s_m0, s_m1, s_g0, s_g1, s_o0, s_o1):
    sem_i = (s_i0, s_i1)
    sem_m = (s_m0, s_m1)
    sem_g = (s_g0, s_g1)
    sem_o = (s_o0, s_o1)
    wid = lax.axis_index("s") * 2 + lax.axis_index("c")
    row0 = wid * G
    nj = D // _LANES

    def issue_idx(r, p):
      pltpu.async_copy(idx_hbm.at[r], idx_v.at[p], sem_i[p])
      pltpu.async_copy(msp_hbm.at[r], msp_v.at[p], sem_m[p])

    def wait_idx(p):
      pltpu.make_async_copy(idx_hbm.at[0], idx_v.at[p], sem_i[p]).wait()
      pltpu.make_async_copy(msp_hbm.at[0], msp_v.at[p], sem_m[p]).wait()

    def issue_gath(p):
      pltpu.async_copy(tab_hbm.at[idx_v.at[p, 0]], rows_v.at[p], sem_g[p])

    def wait_gath(p):
      pltpu.make_async_copy(tab_hbm.at[idx_v.at[p, 0]], rows_v.at[p],
                            sem_g[p]).wait()

    def issue_out(r, p):
      pltpu.async_copy(rows_v.at[p], out_hbm.at[pl.ds(r * _KTOK, _KTOK)],
                       sem_o[p])

    def wait_out(p):
      pltpu.make_async_copy(rows_v.at[p], out_hbm.at[pl.ds(0, _KTOK)],
                            sem_o[p]).wait()

    pltpu.sync_copy(pe0_hbm, pe_v)
    pltpu.sync_copy(dif_hbm, dif_v)
    dvals = [dif_v[0, pl.ds(j * _LANES, _LANES)] for j in range(nj)]

    def compute(p, r):
      li0 = lax.rem(r * _KTOK, L)

      def tok_body(i, li):
        m = msp_v[p, i]
        for j in range(nj):
          sl = pl.ds(j * _LANES, _LANES)
          rows_v[p, i, sl] = rows_v[p, i, sl] + pe_v[li, sl] + m * dvals[j]
        return jnp.where(li == L - 1, 0, li + 1)

      lax.fori_loop(0, _KTOK, tok_body, li0, unroll=4)

    issue_idx(row0, 0)
    issue_idx(row0 + 1, 1)
    wait_idx(0)
    issue_gath(0)

    def body(t, carry):
      for b in range(2):
        p = b
        q = 1 - b
        r = row0 + 2 * t + b
        wait_gath(p)
        if b == 0:
          wait_idx(q)

          @pl.when(t >= 1)
          def _():
            wait_out(q)

          issue_gath(q)
        else:
          @pl.when(t < G // 2 - 1)
          def _():
            wait_idx(q)
            wait_out(q)
            issue_gath(q)
        compute(p, r)
        issue_out(r, p)

        @pl.when(t < G // 2 - 1)
        def _():
          issue_idx(r + 2, p)
      return carry

    lax.fori_loop(0, G // 2, body, 0)
    wait_out(0)
    wait_out(1)

  return k(idx2, msp, token_table, pe0, dif)


def kernel(x, segment_tokens, token_table, segment_table, pe):
  B, L = x.shape
  V, D = token_table.shape
  T = B * L
  NW = 32
  seg = segment_table.astype(jnp.float32)
  pe0 = pe[:L, :] + seg[0][None, :]
  dif = (seg[1] - seg[0]).reshape(1, D)
  NBLK = T // _KTOK
  idx2 = x.astype(jnp.int32).reshape(NBLK, 1, _KTOK)
  m2 = segment_tokens.astype(jnp.float32).reshape(NBLK, _KTOK)
  msp = jnp.broadcast_to(m2[:, :, None], (NBLK, _KTOK, _LANES))
  out = _sc_embed(idx2, msp, token_table, pe0, dif, T, D, L, NW)
  return out.reshape(B, L, D)


# speedup vs baseline: 2.2224x; 2.2224x over previous
import functools

import jax
import jax.numpy as jnp
from jax import lax
from jax.experimental import pallas as pl
from jax.experimental.pallas import tpu as pltpu
from jax.experimental.pallas import tpu_sc as plsc

_LANES = 16
_KTOK = 128


@functools.partial(jax.jit, static_argnums=(3, 4, 5))
def _sc_embed(idx2, token_table, comb, T, D, NW):
  G = (T // _KTOK) // NW
  mesh = plsc.VectorSubcoreMesh(core_axis_name="c", subcore_axis_name="s")

  @functools.partial(
      pl.kernel,
      mesh=mesh,
      out_type=jax.ShapeDtypeStruct((T, D), jnp.float32),
      scratch_types=[
          pltpu.VMEM((2, 2, _KTOK), jnp.int32),
          pltpu.VMEM((2, _KTOK, D), jnp.float32),
          pltpu.VMEM((2, _KTOK, D), jnp.float32),
      ] + [pltpu.SemaphoreType.DMA] * 8,
  )
  def k(idx_hbm, tab_hbm, comb_hbm, out_hbm, idx_v, rows_v, crows_v,
        s_i0, s_i1, s_ga0, s_ga1, s_gb0, s_gb1, s_o0, s_o1):
    sem_i = (s_i0, s_i1)
    sem_ga = (s_ga0, s_ga1)
    sem_gb = (s_gb0, s_gb1)
    sem_o = (s_o0, s_o1)
    wid = lax.axis_index("s") * 2 + lax.axis_index("c")
    row0 = wid * G

    def issue_idx(r, p):
      pltpu.async_copy(idx_hbm.at[r], idx_v.at[p], sem_i[p])

    def wait_idx(p):
      pltpu.make_async_copy(idx_hbm.at[0], idx_v.at[p], sem_i[p]).wait()

    def issue_gath(p):
      pltpu.async_copy(tab_hbm.at[idx_v.at[p, 0]], rows_v.at[p], sem_ga[p])
      pltpu.async_copy(comb_hbm.at[idx_v.at[p, 1]], crows_v.at[p], sem_gb[p])

    def wait_gath(p):
      pltpu.make_async_copy(tab_hbm.at[idx_v.at[p, 0]], rows_v.at[p],
                            sem_ga[p]).wait()
      pltpu.make_async_copy(comb_hbm.at[idx_v.at[p, 1]], crows_v.at[p],
                            sem_gb[p]).wait()

    def issue_out(r, p):
      pltpu.async_copy(rows_v.at[p], out_hbm.at[pl.ds(r * _KTOK, _KTOK)],
                       sem_o[p])

    def wait_out(p):
      pltpu.make_async_copy(rows_v.at[p], out_hbm.at[pl.ds(0, _KTOK)],
                            sem_o[p]).wait()

    def compute(p):
      def add_body(i, c_):
        for j in range(D // _LANES):
          plsc.addupdate(rows_v.at[p, i, pl.ds(j * _LANES, _LANES)],
                         crows_v[p, i, pl.ds(j * _LANES, _LANES)])
        return c_

      lax.fori_loop(0, _KTOK, add_body, 0, unroll=4)

    issue_idx(row0, 0)
    issue_idx(row0 + 1, 1)
    wait_idx(0)
    issue_gath(0)

    def body(t, carry):
      for b in range(2):
        g = 2 * t + b
        p = b
        q = 1 - b
        r = row0 + g
        wait_gath(p)
        if b == 0:
          wait_idx(q)

          @pl.when(t >= 1)
          def _():
            wait_out(q)

          issue_gath(q)

          @pl.when(t < G // 2 - 1)
          def _():
            issue_idx(r + 2, p)
        else:
          @pl.when(t < G // 2 - 1)
          def _():
            wait_idx(q)
            wait_out(q)
            issue_gath(q)
            issue_idx(r + 2, p)
        compute(p)
        issue_out(r, p)
      return carry

    lax.fori_loop(0, G // 2, body, 0)
    wait_out(0)
    wait_out(1)

  return k(idx2, token_table, comb)


def kernel(x, segment_tokens, token_table, segment_table, pe):
  B, L = x.shape
  V, D = token_table.shape
  T = B * L
  NW = 32
  comb = (segment_table.astype(jnp.float32)[:, None, :]
          + pe[:L, :][None, :, :]).reshape(2 * L, D)
  cidx = (segment_tokens.astype(jnp.int32) * L
          + jnp.arange(L, dtype=jnp.int32)[None, :])
  x2 = x.astype(jnp.int32).reshape(T // _KTOK, _KTOK)
  c2 = cidx.reshape(T // _KTOK, _KTOK)
  idx2 = jnp.stack([x2, c2], axis=1)
  out = _sc_embed(idx2, token_table, comb, T, D, NW)
  return out.reshape(B, L, D)
